# baseline (speedup 1.0000x reference)
import functools

import jax
import jax.numpy as jnp
from jax import lax
from jax.experimental import pallas as pl
from jax.experimental.pallas import tpu as pltpu
from jax.experimental.pallas import tpu_sc as plsc

L = 16
CHUNK = 8192
UNROLL = 8


def _sc_call(B, INDIM, OUTDIM, NNZP):
    info = plsc.get_sparse_core_info()
    nw = info.num_cores * info.num_subcores
    assert B % nw == 0
    bpw = B // nw
    n_pairs = NNZP // (2 * CHUNK)
    mesh = plsc.VectorSubcoreMesh(core_axis_name="c", subcore_axis_name="s")

    @functools.partial(
        pl.kernel,
        out_type=jax.ShapeDtypeStruct((B, OUTDIM), jnp.float32),
        mesh=mesh,
        compiler_params=pltpu.CompilerParams(needs_layout_passes=False),
        scratch_types=(
            [pltpu.VMEM((INDIM,), jnp.float32) for _ in range(bpw)]
            + [pltpu.VMEM((OUTDIM,), jnp.float32) for _ in range(bpw)]
            + [pltpu.VMEM((CHUNK,), jnp.int32),
               pltpu.VMEM((CHUNK // 2,), jnp.int32),
               pltpu.VMEM((CHUNK,), jnp.int32),
               pltpu.VMEM((CHUNK // 2,), jnp.int32),
               pltpu.SemaphoreType.DMA,
               pltpu.SemaphoreType.DMA,
               pltpu.SemaphoreType.DMA]
        ),
    )
    def f(in_hbm, w_hbm, bias_hbm, pk_hbm, out_hbm, *scratch):
        inp_v = scratch[:bpw]
        acc_v = scratch[bpw:2 * bpw]
        bufa = scratch[2 * bpw:2 * bpw + 2]
        bufb = scratch[2 * bpw + 2:2 * bpw + 4]
        sem_r, sem_a, sem_b = scratch[2 * bpw + 4:]
        HALF = CHUNK // 2

        def start(c, bufs, sem):
            pk_v, w_v = bufs
            pltpu.async_copy(pk_hbm.at[pl.ds(c * CHUNK, CHUNK)], pk_v, sem)
            pltpu.async_copy(w_hbm.at[pl.ds(c * HALF, HALF)], w_v, sem)

        def drain(bufs, sem):
            pk_v, w_v = bufs
            pltpu.make_async_copy(pk_hbm.at[pl.ds(0, CHUNK)], pk_v, sem).wait()
            pltpu.make_async_copy(w_hbm.at[pl.ds(0, HALF)], w_v, sem).wait()

        def process(bufs):
            pk_v, w_v = bufs

            @plsc.parallel_loop(0, CHUNK // (2 * L), unroll=UNROLL)
            def body(t):
                wp = w_v[pl.ds(t * L, L)]
                wlo = plsc.bitcast(lax.shift_left(wp, 16), jnp.float32)
                whi = plsc.bitcast(wp & jnp.int32(-65536), jnp.float32)
                for half, wv in ((0, wlo), (1, whi)):
                    pk = pk_v[pl.ds(t * 2 * L + half * L, L)]
                    ii = pk & 0xFFFF
                    io = lax.shift_right_logical(pk, 16)
                    for b in range(bpw):
                        x = plsc.load_gather(inp_v[b], [ii]) * wv
                        plsc.addupdate_scatter(acc_v[b], [io], x)

        wid = lax.axis_index("s") * info.num_cores + lax.axis_index("c")
        b0 = wid * bpw

        start(0, bufa, sem_a)
        row_cps = []
        for b in range(bpw):
            row_cps.append(pltpu.async_copy(in_hbm.at[b0 + b], inp_v[b], sem_r))
            row_cps.append(pltpu.async_copy(bias_hbm, acc_v[b], sem_r))
        for cp in row_cps:
            cp.wait()

        def pair_body(p, _):
            start(2 * p + 1, bufb, sem_b)
            drain(bufa, sem_a)
            process(bufa)

            @pl.when(p + 1 < n_pairs)
            def _():
                start(2 * p + 2, bufa, sem_a)

            drain(bufb, sem_b)
            process(bufb)
            return ()

        lax.fori_loop(0, n_pairs, pair_body, ())

        for b in range(bpw):
            pltpu.sync_copy(acc_v[b], out_hbm.at[b0 + b])

    return f


def kernel(_input, weight, bias, ind_in, ind_out):
    B, INDIM = _input.shape
    OUTDIM = bias.shape[0]
    nnz = weight.shape[0]
    step = 2 * CHUNK
    nnzp = ((nnz + step - 1) // step) * step
    pad = nnzp - nnz
    packed = jnp.bitwise_or(jnp.left_shift(ind_out, 16), ind_in)
    if pad:
        spread = jnp.arange(pad, dtype=jnp.int32)
        weight = jnp.concatenate([weight, jnp.zeros((pad,), weight.dtype)])
        pad_pk = jnp.bitwise_or(
            jnp.left_shift(spread % OUTDIM, 16), spread % INDIM)
        packed = jnp.concatenate([packed, pad_pk])
    wb = weight.astype(jnp.bfloat16).reshape(-1, 2, L).transpose(0, 2, 1)
    w32 = jax.lax.bitcast_convert_type(wb, jnp.int32).reshape(-1)
    f = _sc_call(B, INDIM, OUTDIM, nnzp)
    return f(_input, w32, bias, packed)


# speedup vs baseline: 1.4768x; 1.4768x over previous
import functools

import jax
import jax.numpy as jnp
from jax import lax
from jax.experimental import pallas as pl
from jax.experimental.pallas import tpu as pltpu
from jax.experimental.pallas import tpu_sc as plsc

L = 16
CHUNK = 4096
UNROLL = 8


def _sc_call(B, INDIM, OUTDIM, NNZP):
    info = plsc.get_sparse_core_info()
    nw = info.num_cores * info.num_subcores
    assert B % nw == 0
    bpw = B // nw
    n_pairs = NNZP // (2 * CHUNK)
    mesh = plsc.VectorSubcoreMesh(core_axis_name="c", subcore_axis_name="s")

    @functools.partial(
        pl.kernel,
        out_type=jax.ShapeDtypeStruct((B, OUTDIM), jnp.float32),
        mesh=mesh,
        compiler_params=pltpu.CompilerParams(needs_layout_passes=False),
        scratch_types=(
            [pltpu.VMEM((INDIM,), jnp.float32) for _ in range(bpw)]
            + [pltpu.VMEM((OUTDIM,), jnp.float32) for _ in range(bpw)]
            + [pltpu.VMEM((CHUNK,), jnp.int32),
               pltpu.VMEM((CHUNK,), jnp.float32),
               pltpu.VMEM((CHUNK,), jnp.int32),
               pltpu.VMEM((CHUNK,), jnp.float32),
               pltpu.SemaphoreType.DMA,
               pltpu.SemaphoreType.DMA,
               pltpu.SemaphoreType.DMA]
        ),
    )
    def f(in_hbm, w_hbm, bias_hbm, pk_hbm, out_hbm, *scratch):
        inp_v = scratch[:bpw]
        acc_v = scratch[bpw:2 * bpw]
        bufa = scratch[2 * bpw:2 * bpw + 2]
        bufb = scratch[2 * bpw + 2:2 * bpw + 4]
        sem_r, sem_a, sem_b = scratch[2 * bpw + 4:]
        hbms = (pk_hbm, w_hbm)

        def start(c, bufs, sem):
            for hbm, v in zip(hbms, bufs):
                pltpu.async_copy(hbm.at[pl.ds(c * CHUNK, CHUNK)], v, sem)

        def drain(bufs, sem):
            for hbm, v in zip(hbms, bufs):
                pltpu.make_async_copy(hbm.at[pl.ds(0, CHUNK)], v, sem).wait()

        def process(bufs):
            pk_v, w_v = bufs

            @plsc.parallel_loop(0, CHUNK // L, unroll=UNROLL)
            def body(j):
                pk = pk_v[pl.ds(j * L, L)]
                ii = pk & 0xFFFF
                io = lax.shift_right_logical(pk, 16)
                wv = w_v[pl.ds(j * L, L)]
                for b in range(bpw):
                    x = plsc.load_gather(inp_v[b], [ii]) * wv
                    plsc.addupdate_scatter(acc_v[b], [io], x)

        wid = lax.axis_index("s") * info.num_cores + lax.axis_index("c")
        b0 = wid * bpw

        start(0, bufa, sem_a)
        row_cps = []
        for b in range(bpw):
            row_cps.append(pltpu.async_copy(in_hbm.at[b0 + b], inp_v[b], sem_r))
            row_cps.append(pltpu.async_copy(bias_hbm, acc_v[b], sem_r))
        for cp in row_cps:
            cp.wait()

        def pair_body(p, _):
            start(2 * p + 1, bufb, sem_b)
            drain(bufa, sem_a)
            process(bufa)

            @pl.when(p + 1 < n_pairs)
            def _():
                start(2 * p + 2, bufa, sem_a)

            drain(bufb, sem_b)
            process(bufb)
            return ()

        lax.fori_loop(0, n_pairs, pair_body, ())

        for b in range(bpw):
            pltpu.sync_copy(acc_v[b], out_hbm.at[b0 + b])

    return f


def kernel(_input, weight, bias, ind_in, ind_out):
    B, INDIM = _input.shape
    OUTDIM = bias.shape[0]
    nnz = weight.shape[0]
    step = 2 * CHUNK
    nnzp = ((nnz + step - 1) // step) * step
    pad = nnzp - nnz
    packed = jnp.bitwise_or(jnp.left_shift(ind_out, 16), ind_in)
    if pad:
        spread = jnp.arange(pad, dtype=jnp.int32)
        weight = jnp.concatenate([weight, jnp.zeros((pad,), weight.dtype)])
        pad_pk = jnp.bitwise_or(
            jnp.left_shift(spread % OUTDIM, 16), spread % INDIM)
        packed = jnp.concatenate([packed, pad_pk])
    f = _sc_call(B, INDIM, OUTDIM, nnzp)
    return f(_input, weight, bias, packed)
